# Initial kernel scaffold; baseline (speedup 1.0000x reference)
import jax
import jax.numpy as jnp
from jax.experimental import pallas as pl


def kernel(input_ids, expr_bins, protein_emb, expr_table):
    raise NotImplementedError("write your pallas kernel here")


# speedup vs baseline: 1.7299x; 1.7299x over previous
import functools

import jax
import jax.numpy as jnp
from jax import lax
from jax.experimental import pallas as pl
from jax.experimental.pallas import tpu as pltpu
from jax.experimental.pallas import tpu_sc as plsc

NC = 2
NS = 16
NW = NC * NS


@functools.lru_cache(maxsize=None)
def _make_kernel(n_idx, gene_d, expr_d):
    gb = n_idx // NW
    gc = 64
    ec = 128
    g_steps = gb // gc
    e_steps = gb // ec
    mesh = plsc.VectorSubcoreMesh(core_axis_name="c", subcore_axis_name="s")

    @functools.partial(
        pl.kernel,
        mesh=mesh,
        out_type=(
            jax.ShapeDtypeStruct((n_idx, gene_d), jnp.float32),
            jax.ShapeDtypeStruct((n_idx, expr_d), jnp.float32),
        ),
        scratch_types=[
            pltpu.VMEM((gc,), jnp.int32),
            pltpu.VMEM((ec,), jnp.int32),
            pltpu.VMEM((gc, gene_d), jnp.float32),
            pltpu.VMEM((ec, expr_d), jnp.float32),
            pltpu.SemaphoreType.DMA,
        ],
    )
    def emb_kernel(ids_hbm, bins_hbm, ptab_hbm, etab_hbm,
                   gene_out, expr_out, gidx_v, eidx_v, gbuf, ebuf, sem):
        wid = lax.axis_index("s") * NC + lax.axis_index("c")
        wbase = wid * gb

        def gene_body(i, carry):
            base = wbase + i * gc
            pltpu.sync_copy(ids_hbm.at[pl.ds(base, gc)], gidx_v)
            pltpu.async_copy(ptab_hbm.at[gidx_v], gbuf, sem).wait()
            pltpu.sync_copy(gbuf, gene_out.at[pl.ds(base, gc)])
            return carry

        lax.fori_loop(0, g_steps, gene_body, 0, unroll=False)

        def expr_body(i, carry):
            base = wbase + i * ec
            pltpu.sync_copy(bins_hbm.at[pl.ds(base, ec)], eidx_v)
            pltpu.async_copy(etab_hbm.at[eidx_v], ebuf, sem).wait()
            pltpu.sync_copy(ebuf, expr_out.at[pl.ds(base, ec)])
            return carry

        lax.fori_loop(0, e_steps, expr_body, 0, unroll=False)

    return emb_kernel


def kernel(input_ids, expr_bins, protein_emb, expr_table):
    b, l = input_ids.shape
    n = b * l
    ids = input_ids.reshape(n).astype(jnp.int32)
    bins = expr_bins.reshape(n).astype(jnp.int32)
    gene_d = protein_emb.shape[1]
    expr_d = expr_table.shape[1]
    emb = _make_kernel(n, gene_d, expr_d)
    gene, expr = emb(ids, bins, protein_emb, expr_table)
    return gene.reshape(b, l, gene_d), expr.reshape(b, l, expr_d)


# speedup vs baseline: 1.8042x; 1.0429x over previous
import functools

import jax
import jax.numpy as jnp
from jax import lax
from jax.experimental import pallas as pl
from jax.experimental.pallas import tpu as pltpu
from jax.experimental.pallas import tpu_sc as plsc

NC = 2
NS = 16
NW = NC * NS


@functools.lru_cache(maxsize=None)
def _make_kernel(n_idx, gene_d, expr_d):
    gb = n_idx // NW
    gc = 32
    ec = 64
    g_steps = gb // gc
    e_steps = gb // ec
    mesh = plsc.VectorSubcoreMesh(core_axis_name="c", subcore_axis_name="s")

    @functools.partial(
        pl.kernel,
        mesh=mesh,
        out_type=(
            jax.ShapeDtypeStruct((n_idx, gene_d), jnp.float32),
            jax.ShapeDtypeStruct((n_idx, expr_d), jnp.float32),
        ),
        scratch_types=[
            pltpu.VMEM((gb,), jnp.int32),
            pltpu.VMEM((gb,), jnp.int32),
            pltpu.VMEM((gc, gene_d), jnp.float32),
            pltpu.VMEM((gc, gene_d), jnp.float32),
            pltpu.VMEM((ec, expr_d), jnp.float32),
            pltpu.VMEM((ec, expr_d), jnp.float32),
            pltpu.SemaphoreType.DMA,
            pltpu.SemaphoreType.DMA,
            pltpu.SemaphoreType.DMA,
            pltpu.SemaphoreType.DMA,
        ],
    )
    def emb_kernel(ids_hbm, bins_hbm, ptab_hbm, etab_hbm,
                   gene_out, expr_out, gidx_v, eidx_v,
                   gb0, gb1, eb0, eb1, s0, s1, w0, w1):
        wid = lax.axis_index("s") * NC + lax.axis_index("c")
        wbase = wid * gb
        pltpu.sync_copy(ids_hbm.at[pl.ds(wbase, gb)], gidx_v)
        pltpu.sync_copy(bins_hbm.at[pl.ds(wbase, gb)], eidx_v)

        def phase(idx_v, tab, out, buf0, buf1, chunk, steps):
            def body(j, carry):
                c0 = j * 4

                def gather(c, buf, sem):
                    return pltpu.async_copy(
                        tab.at[idx_v.at[pl.ds(c * chunk, chunk)]], buf, sem)

                def write(c, buf, sem):
                    return pltpu.async_copy(
                        buf, out.at[pl.ds(wbase + c * chunk, chunk)], sem)

                g0 = gather(c0 + 0, buf0, s0)
                g1 = gather(c0 + 1, buf1, s1)
                g0.wait()
                wr0 = write(c0 + 0, buf0, w0)
                g1.wait()
                wr1 = write(c0 + 1, buf1, w1)
                wr0.wait()
                g2 = gather(c0 + 2, buf0, s0)
                wr1.wait()
                g3 = gather(c0 + 3, buf1, s1)
                g2.wait()
                wr2 = write(c0 + 2, buf0, w0)
                g3.wait()
                wr3 = write(c0 + 3, buf1, w1)
                wr2.wait()
                wr3.wait()
                return carry

            lax.fori_loop(0, steps // 4, body, 0, unroll=False)

        phase(gidx_v, ptab_hbm, gene_out, gb0, gb1, gc, g_steps)
        phase(eidx_v, etab_hbm, expr_out, eb0, eb1, ec, e_steps)

    return emb_kernel


def kernel(input_ids, expr_bins, protein_emb, expr_table):
    b, l = input_ids.shape
    n = b * l
    ids = input_ids.reshape(n).astype(jnp.int32)
    bins = expr_bins.reshape(n).astype(jnp.int32)
    gene_d = protein_emb.shape[1]
    expr_d = expr_table.shape[1]
    emb = _make_kernel(n, gene_d, expr_d)
    gene, expr = emb(ids, bins, protein_emb, expr_table)
    return gene.reshape(b, l, gene_d), expr.reshape(b, l, expr_d)
